# baseline (speedup 1.0000x reference)
import functools
import jax
import jax.numpy as jnp
from jax import lax
from jax.experimental import pallas as pl
from jax.experimental.pallas import tpu as pltpu
from jax.experimental.pallas import tpu_sc as plsc

SCORE_THR = 0.05
IOU_THR = 0.3
MAX_OUT = 100
N = 5000
C = 21
B = 2
NC = 2
NS = 16
L = 16
NPAD = 5120
PT = NPAD // NS
G = PT // L
OUTP = 112
ZROW = B * NPAD


def _sc_body(del_h, prop_h, log_h, ltab_h, orow_h, gath_h,
             dl, pr, lg, by1, bx1, by2, bx2, bcl, bsc,
             sy1, sx1, sy2, sx2, ssc, sky, cend,
             orow, lrow, idxb, tvec, rbuf, shared, sem):
    c = lax.axis_index("c")
    s = lax.axis_index("s")
    base = s * PT
    c_off = c * NPAD

    iota = lax.iota(jnp.int32, L)
    iotaf = iota.astype(jnp.float32)
    zv = jnp.zeros((L,), jnp.float32)
    zi = jnp.zeros((L,), jnp.int32)

    for k in range(4):
        pltpu.sync_copy(del_h.at[c * 4 + k, pl.ds(base, PT)], dl.at[k])
        pltpu.sync_copy(prop_h.at[c * 4 + k, pl.ds(base, PT)], pr.at[k])
    for k in range(C):
        pltpu.sync_copy(log_h.at[c * C + k, pl.ds(base, PT)], lg.at[k])

    def init(j, _):
        orow[pl.ds(j * L, L)] = zv
        return 0
    lax.fori_loop(0, (OUTP * L) // L, init, 0)

    def init2(j, _):
        idxb[pl.ds(j * L, L)] = zi + ZROW
        return 0
    lax.fori_loop(0, OUTP // L, init2, 0)

    def prep(g, _):
        sl = pl.ds(g * L, L)
        d0 = dl[0, sl]; d1 = dl[1, sl]; d2 = dl[2, sl]; d3 = dl[3, sl]
        p0 = pr[0, sl]; p1 = pr[1, sl]; p2 = pr[2, sl]; p3 = pr[3, sl]
        h = p2 - p0
        w = p3 - p1
        cy = p0 + 0.5 * h + d0 * h
        cx = p1 + 0.5 * w + d1 * w
        h2 = h * jnp.exp(d2)
        w2 = w * jnp.exp(d3)
        y1 = cy - 0.5 * h2
        x1 = cx - 0.5 * w2
        y2 = cy + 0.5 * h2
        x2 = cx + 0.5 * w2
        m1 = lg[1, sl]
        cls = zi + 1
        for cc in range(2, C):
            v = lg[cc, sl]
            upd = v > m1
            m1 = jnp.where(upd, v, m1)
            cls = jnp.where(upd, cc, cls)
        m_all = jnp.maximum(lg[0, sl], m1)
        z = zv
        for cc in range(C):
            z = z + jnp.exp(lg[cc, sl] - m_all)
        sc = jnp.exp(m1 - m_all) / z
        gi = base + g * L + iota
        ok = (sc > SCORE_THR) & (gi < N)
        by1[sl] = y1
        bx1[sl] = x1
        by2[sl] = y2
        bx2[sl] = x2
        bcl[sl] = cls * 8192 + gi
        bsc[sl] = jnp.where(ok, sc, -1.0)
        return 0

    lax.fori_loop(0, G, prep, 0)

    off = jnp.asarray(0, jnp.int32)
    ends = []
    for ccls in range(1, C):
        def cbody(g, off, _ccls=ccls):
            sl = pl.ds(g * L, L)
            kv = bcl[sl]
            msk = (kv >> 13) == _ccls
            dsl = pl.ds(off, L)
            plsc.store_compressed(sy1.at[dsl], by1[sl], mask=msk)
            plsc.store_compressed(sx1.at[dsl], bx1[sl], mask=msk)
            plsc.store_compressed(sy2.at[dsl], by2[sl], mask=msk)
            plsc.store_compressed(sx2.at[dsl], bx2[sl], mask=msk)
            plsc.store_compressed(ssc.at[dsl], bsc[sl], mask=msk)
            plsc.store_compressed(sky.at[dsl], kv, mask=msk)
            return off + plsc.all_reduce_population_count(msk)[0]
        off = lax.fori_loop(0, G, cbody, off)
        ends.append(off)
    cv0 = zi
    for cidx in range(1, 16):
        cv0 = jnp.where(iota == cidx, ends[cidx - 1], cv0)
    cv1 = zi + ends[19]
    for cidx in range(16, 21):
        cv1 = jnp.where(iota == (cidx - 16), ends[cidx - 1], cv1)
    cend[pl.ds(0, L)] = cv0
    cend[pl.ds(L, L)] = cv1

    BIGK = 2 ** 28

    def nms_it(it, carry):
        wy1, wx1, wy2, wx2, wclv, warea, wkeyv = carry

        stv = plsc.load_gather(cend, [wclv - 1])
        env = plsc.load_gather(cend, [wclv])
        g0 = stv[0] >> 4
        g1 = (env[0] + 15) >> 4

        def supp(g, _):
            sl = pl.ds(g * L, L)
            y1 = sy1[sl]; x1 = sx1[sl]; y2 = sy2[sl]; x2 = sx2[sl]
            yy1 = jnp.maximum(wy1, y1)
            xx1 = jnp.maximum(wx1, x1)
            yy2 = jnp.minimum(wy2, y2)
            xx2 = jnp.minimum(wx2, x2)
            inter = jnp.maximum(yy2 - yy1, 0.0) * jnp.maximum(xx2 - xx1, 0.0)
            ar = (y2 - y1) * (x2 - x1)
            iou = inter / (warea + ar - inter + 1e-8)
            kv = sky[sl]
            kill = (((kv >> 13) == wclv) & (iou > IOU_THR)) | (kv == wkeyv)
            ssc[sl] = jnp.where(kill, -1.0, ssc[sl])
            return 0

        lax.fori_loop(g0, g1, supp, 0)

        NACC = 4
        accs = [(zv - 2.0, zi + BIGK) for _ in range(NACC)]
        for g in range(G):
            sl = pl.ds(g * L, L)
            sc2 = ssc[sl]
            kv = sky[sl]
            bv, bk = accs[g % NACC]
            upd = (sc2 > bv) | ((sc2 == bv) & (kv < bk))
            accs[g % NACC] = (jnp.where(upd, sc2, bv), jnp.where(upd, kv, bk))
        while len(accs) > 1:
            (v1, k1), (v2, k2) = accs[0], accs[1]
            upd = (v2 > v1) | ((v2 == v1) & (k2 < k1))
            accs = accs[2:] + [(jnp.where(upd, v2, v1), jnp.where(upd, k2, k1))]
        bv, bk = accs[0]
        m = jnp.max(bv)
        lkey = zi + jnp.min(jnp.where(bv == m, bk, BIGK))
        liv = (lkey & 8191) - base
        slot = jnp.where(iota == 0, zv + m,
               jnp.where(iota == 1, plsc.bitcast(lkey, jnp.float32),
               jnp.where(iota == 2, plsc.load_gather(by1, [liv]),
               jnp.where(iota == 3, plsc.load_gather(bx1, [liv]),
               jnp.where(iota == 4, plsc.load_gather(by2, [liv]),
                         plsc.load_gather(bx2, [liv]))))))
        tvec[...] = slot
        par = it & 1
        pltpu.sync_copy(tvec, shared.at[par, pl.ds(s * L, L)])
        plsc.subcore_barrier()
        pltpu.sync_copy(shared.at[par], rbuf)

        vals = plsc.load_gather(rbuf, [iota * L])
        keys = plsc.bitcast(plsc.load_gather(rbuf, [iota * L + 1]), jnp.int32)
        gm = jnp.max(vals)
        nkey = zi + jnp.min(jnp.where(vals == gm, keys, BIGK))
        wlane = zi + plsc.all_reduce_ffs(keys == nkey)

        def fw(f):
            return plsc.load_gather(rbuf, [wlane * L + f])

        ny1 = fw(2); nx1 = fw(3); ny2 = fw(4); nx2 = fw(5)
        ncl = nkey >> 13
        nwidx = nkey & 8191
        gmv = zv + gm
        validv = gmv > 0.0
        narea = (ny2 - ny1) * (nx2 - nx1)

        tagv = jnp.where(validv, zv + 1.0, zv)
        gidxf = jnp.where(validv, nwidx + c_off, zi + ZROW).astype(jnp.float32)
        nclf = ncl.astype(jnp.float32)
        row = jnp.where(iota == 0, jnp.where(validv, ny1, zv),
              jnp.where(iota == 1, jnp.where(validv, nx1, zv),
              jnp.where(iota == 2, jnp.where(validv, ny2, zv),
              jnp.where(iota == 3, jnp.where(validv, nx2, zv),
              jnp.where(iota == 4, tagv,
              jnp.where(iota == 5, jnp.where(validv, gmv, zv),
              jnp.where(iota == 6, tagv,
              jnp.where(iota == 7, jnp.where(validv, nclf, zv),
              jnp.where(iota == 8, tagv,
              jnp.where(iota == 9, gidxf, zv))))))))))
        orow[pl.ds(it * L, L)] = row
        return ny1, nx1, ny2, nx2, ncl, narea, nkey

    lax.fori_loop(0, MAX_OUT, nms_it,
                  (zv, zv, zv, zv, zi + 21, zv, zi - 1))

    @pl.when(s == 0)
    def _():
        for j in range(OUTP // L):
            gidx = plsc.load_gather(orow, [(j * L + iota) * L + 9])
            idxb[pl.ds(j * L, L)] = gidx.astype(jnp.int32)
        pltpu.async_copy(ltab_h.at[idxb], lrow, sem).wait()
        pltpu.sync_copy(orow, orow_h.at[c])
        pltpu.sync_copy(lrow, gath_h.at[c])


@functools.cache
def _build_sc_nms():
  mesh = plsc.VectorSubcoreMesh(core_axis_name="c", subcore_axis_name="s",
                                num_cores=NC, num_subcores=NS)
  return functools.partial(
    pl.kernel,
    out_type=(jax.ShapeDtypeStruct((B, OUTP * L), jnp.float32),
              jax.ShapeDtypeStruct((B, OUTP, 32), jnp.float32)),
    mesh=mesh,
    compiler_params=pltpu.CompilerParams(use_tc_tiling_on_sc=False,
                                         needs_layout_passes=False),
    scratch_types=[
        pltpu.VMEM((4, PT), jnp.float32),
        pltpu.VMEM((4, PT), jnp.float32),
        pltpu.VMEM((C, PT), jnp.float32),
        pltpu.VMEM((PT,), jnp.float32),
        pltpu.VMEM((PT,), jnp.float32),
        pltpu.VMEM((PT,), jnp.float32),
        pltpu.VMEM((PT,), jnp.float32),
        pltpu.VMEM((PT,), jnp.int32),
        pltpu.VMEM((PT,), jnp.float32),
        pltpu.VMEM((PT + L,), jnp.float32),
        pltpu.VMEM((PT + L,), jnp.float32),
        pltpu.VMEM((PT + L,), jnp.float32),
        pltpu.VMEM((PT + L,), jnp.float32),
        pltpu.VMEM((PT + L,), jnp.float32),
        pltpu.VMEM((PT + L,), jnp.int32),
        pltpu.VMEM((2 * L,), jnp.int32),
        pltpu.VMEM((OUTP * L,), jnp.float32),
        pltpu.VMEM((OUTP, 32), jnp.float32),
        pltpu.VMEM((OUTP,), jnp.int32),
        pltpu.VMEM((L,), jnp.float32),
        pltpu.VMEM((NS * L,), jnp.float32),
        pltpu.VMEM_SHARED((2, NS * L), jnp.float32),
        pltpu.SemaphoreType.DMA,
    ],
  )(_sc_body)


@jax.jit
def kernel(deltas, class_logits, proposals):
    pad_n = ((0, 0), (0, 0), (0, NPAD - N))
    d_t = jnp.pad(jnp.transpose(deltas, (0, 2, 1)), pad_n).reshape(B * 4, NPAD)
    p_t = jnp.pad(jnp.transpose(proposals[..., :4], (0, 2, 1)), pad_n).reshape(B * 4, NPAD)
    l_t = jnp.pad(jnp.transpose(class_logits, (0, 2, 1)), pad_n).reshape(B * C, NPAD)
    ltab = jnp.zeros((B * NPAD + 8, 32), jnp.float32)
    ltab = ltab.at[:B * NPAD, :C].set(
        jnp.pad(class_logits, ((0, 0), (0, NPAD - N), (0, 0))).reshape(B * NPAD, C))

    orow_o, gath_o = _build_sc_nms()(d_t, p_t, l_t, ltab)
    orow = orow_o.reshape(B, OUTP, L)
    boxes_out = orow[:, :MAX_OUT, 0:5]
    scores_out = orow[:, :MAX_OUT, 5:7]
    ids_out = orow[:, :MAX_OUT, 7:9].astype(jnp.int32)
    logits_out = jnp.concatenate(
        [gath_o[:, :MAX_OUT, :C], orow[:, :MAX_OUT, 4:5]], axis=-1)
    return boxes_out, scores_out, ids_out, logits_out


# speedup vs baseline: 1.2725x; 1.2725x over previous
import functools
import jax
import jax.numpy as jnp
from jax import lax
from jax.experimental import pallas as pl
from jax.experimental.pallas import tpu as pltpu
from jax.experimental.pallas import tpu_sc as plsc

SCORE_THR = 0.05
IOU_THR = 0.3
MAX_OUT = 100
N = 5000
C = 21
B = 2
NC = 2
NS = 16
L = 16
NPAD = 5120
PT = NPAD // NS
G = PT // L
OUTP = 112
ZROW = B * NPAD


def _sc_body(del_h, prop_h, log_h, ltab_h, orow_h, gath_h,
             dl, pr, lg, by1, bx1, by2, bx2, bcl, bsc,
             orow, lrow, idxb, tvec, rbuf, shared, sem):
    c = lax.axis_index("c")
    s = lax.axis_index("s")
    base = s * PT
    c_off = c * NPAD

    iota = lax.iota(jnp.int32, L)
    iotaf = iota.astype(jnp.float32)
    zv = jnp.zeros((L,), jnp.float32)
    zi = jnp.zeros((L,), jnp.int32)

    copies = []
    for k in range(4):
        copies.append(pltpu.make_async_copy(
            del_h.at[c * 4 + k, pl.ds(base, PT)], dl.at[k], sem))
        copies.append(pltpu.make_async_copy(
            prop_h.at[c * 4 + k, pl.ds(base, PT)], pr.at[k], sem))
    for k in range(C):
        copies.append(pltpu.make_async_copy(
            log_h.at[c * C + k, pl.ds(base, PT)], lg.at[k], sem))
    for cp in copies:
        cp.start()
    for cp in copies:
        cp.wait()

    def init(j, _):
        orow[pl.ds(j * L, L)] = zv
        return 0
    lax.fori_loop(0, (OUTP * L) // L, init, 0)

    def init2(j, _):
        idxb[pl.ds(j * L, L)] = zi + ZROW
        return 0
    lax.fori_loop(0, OUTP // L, init2, 0)

    def prep(g, _):
        sl = pl.ds(g * L, L)
        d0 = dl[0, sl]; d1 = dl[1, sl]; d2 = dl[2, sl]; d3 = dl[3, sl]
        p0 = pr[0, sl]; p1 = pr[1, sl]; p2 = pr[2, sl]; p3 = pr[3, sl]
        h = p2 - p0
        w = p3 - p1
        cy = p0 + 0.5 * h + d0 * h
        cx = p1 + 0.5 * w + d1 * w
        h2 = h * jnp.exp(d2)
        w2 = w * jnp.exp(d3)
        y1 = cy - 0.5 * h2
        x1 = cx - 0.5 * w2
        y2 = cy + 0.5 * h2
        x2 = cx + 0.5 * w2
        m1 = lg[1, sl]
        cls = zi + 1
        for cc in range(2, C):
            v = lg[cc, sl]
            upd = v > m1
            m1 = jnp.where(upd, v, m1)
            cls = jnp.where(upd, cc, cls)
        m_all = jnp.maximum(lg[0, sl], m1)
        z = zv
        for cc in range(C):
            z = z + jnp.exp(lg[cc, sl] - m_all)
        sc = jnp.exp(m1 - m_all) / z
        gi = base + g * L + iota
        ok = (sc > SCORE_THR) & (gi < N)
        by1[sl] = y1
        bx1[sl] = x1
        by2[sl] = y2
        bx2[sl] = x2
        bcl[sl] = cls * 8192 + gi
        bsc[sl] = jnp.where(ok, sc, -1.0)
        return 0

    lax.fori_loop(0, G, prep, 0)

    BIGK = 2 ** 28

    def nms_it(it, carry):
        wy1, wx1, wy2, wx2, wclv, warea, wkeyv = carry

        NACC = 4
        accs = [(zv - 2.0, zi + BIGK) for _ in range(NACC)]
        for g in range(G):
            sl = pl.ds(g * L, L)
            y1 = by1[sl]; x1 = bx1[sl]; y2 = by2[sl]; x2 = bx2[sl]
            yy1 = jnp.maximum(wy1, y1)
            xx1 = jnp.maximum(wx1, x1)
            yy2 = jnp.minimum(wy2, y2)
            xx2 = jnp.minimum(wx2, x2)
            inter = jnp.maximum(yy2 - yy1, 0.0) * jnp.maximum(xx2 - xx1, 0.0)
            ar = (y2 - y1) * (x2 - x1)
            iou = inter / (warea + ar - inter + 1e-8)
            kv = bcl[sl]
            kill = (((kv >> 13) == wclv) & (iou > IOU_THR)) | (kv == wkeyv)
            sc2 = jnp.where(kill, -1.0, bsc[sl])
            bsc[sl] = sc2
            bv, bk = accs[g % NACC]
            upd = (sc2 > bv) | ((sc2 == bv) & (kv < bk))
            accs[g % NACC] = (jnp.where(upd, sc2, bv), jnp.where(upd, kv, bk))
        while len(accs) > 1:
            (v1, k1), (v2, k2) = accs[0], accs[1]
            upd = (v2 > v1) | ((v2 == v1) & (k2 < k1))
            accs = accs[2:] + [(jnp.where(upd, v2, v1), jnp.where(upd, k2, k1))]
        bv, bk = accs[0]
        m = jnp.max(bv)
        lkey = zi + jnp.min(jnp.where(bv == m, bk, BIGK))
        liv = (lkey & 8191) - base
        slot = jnp.where(iota == 0, zv + m,
               jnp.where(iota == 1, plsc.bitcast(lkey, jnp.float32),
               jnp.where(iota == 2, plsc.load_gather(by1, [liv]),
               jnp.where(iota == 3, plsc.load_gather(bx1, [liv]),
               jnp.where(iota == 4, plsc.load_gather(by2, [liv]),
                         plsc.load_gather(bx2, [liv]))))))
        tvec[...] = slot
        par = it & 1
        pltpu.sync_copy(tvec, shared.at[par, pl.ds(s * L, L)])
        plsc.subcore_barrier()
        pltpu.sync_copy(shared.at[par], rbuf)

        vals = plsc.load_gather(rbuf, [iota * L])
        keys = plsc.bitcast(plsc.load_gather(rbuf, [iota * L + 1]), jnp.int32)
        gm = jnp.max(vals)
        nkey = zi + jnp.min(jnp.where(vals == gm, keys, BIGK))
        wlane = zi + plsc.all_reduce_ffs(keys == nkey)

        def fw(f):
            return plsc.load_gather(rbuf, [wlane * L + f])

        ny1 = fw(2); nx1 = fw(3); ny2 = fw(4); nx2 = fw(5)
        ncl = nkey >> 13
        nwidx = nkey & 8191
        gmv = zv + gm
        validv = gmv > 0.0
        narea = (ny2 - ny1) * (nx2 - nx1)

        tagv = jnp.where(validv, zv + 1.0, zv)
        gidxf = jnp.where(validv, nwidx + c_off, zi + ZROW).astype(jnp.float32)
        nclf = ncl.astype(jnp.float32)
        mk = [jnp.where(iota == j, zv + 1.0, zv) for j in range(10)]
        row = (((mk[0] * ny1 + mk[1] * nx1) + (mk[2] * ny2 + mk[3] * nx2))
               + ((mk[5] * gmv + mk[7] * nclf) * 1.0)) * tagv
        row = row + (mk[4] + mk[6] + mk[8]) * tagv + mk[9] * gidxf
        orow[pl.ds(it * L, L)] = row
        return ny1, nx1, ny2, nx2, ncl, narea, nkey

    lax.fori_loop(0, MAX_OUT, nms_it,
                  (zv, zv, zv, zv, zi - 1, zv, zi - 1))

    @pl.when(s == 0)
    def _():
        for j in range(OUTP // L):
            gidx = plsc.load_gather(orow, [(j * L + iota) * L + 9])
            idxb[pl.ds(j * L, L)] = gidx.astype(jnp.int32)
        pltpu.async_copy(ltab_h.at[idxb], lrow, sem).wait()
        pltpu.sync_copy(orow, orow_h.at[c])
        pltpu.sync_copy(lrow, gath_h.at[c])


@functools.cache
def _build_sc_nms():
  mesh = plsc.VectorSubcoreMesh(core_axis_name="c", subcore_axis_name="s",
                                num_cores=NC, num_subcores=NS)
  return functools.partial(
    pl.kernel,
    out_type=(jax.ShapeDtypeStruct((B, OUTP * L), jnp.float32),
              jax.ShapeDtypeStruct((B, OUTP, 32), jnp.float32)),
    mesh=mesh,
    compiler_params=pltpu.CompilerParams(use_tc_tiling_on_sc=False,
                                         needs_layout_passes=False),
    scratch_types=[
        pltpu.VMEM((4, PT), jnp.float32),
        pltpu.VMEM((4, PT), jnp.float32),
        pltpu.VMEM((C, PT), jnp.float32),
        pltpu.VMEM((PT,), jnp.float32),
        pltpu.VMEM((PT,), jnp.float32),
        pltpu.VMEM((PT,), jnp.float32),
        pltpu.VMEM((PT,), jnp.float32),
        pltpu.VMEM((PT,), jnp.int32),
        pltpu.VMEM((PT,), jnp.float32),
        pltpu.VMEM((OUTP * L,), jnp.float32),
        pltpu.VMEM((OUTP, 32), jnp.float32),
        pltpu.VMEM((OUTP,), jnp.int32),
        pltpu.VMEM((L,), jnp.float32),
        pltpu.VMEM((NS * L,), jnp.float32),
        pltpu.VMEM_SHARED((2, NS * L), jnp.float32),
        pltpu.SemaphoreType.DMA,
    ],
  )(_sc_body)


@jax.jit
def kernel(deltas, class_logits, proposals):
    pad_n = ((0, 0), (0, 0), (0, NPAD - N))
    d_t = jnp.pad(jnp.transpose(deltas, (0, 2, 1)), pad_n).reshape(B * 4, NPAD)
    p_t = jnp.pad(jnp.transpose(proposals[..., :4], (0, 2, 1)), pad_n).reshape(B * 4, NPAD)
    l_t = jnp.pad(jnp.transpose(class_logits, (0, 2, 1)), pad_n).reshape(B * C, NPAD)
    ltab = jnp.zeros((B * NPAD + 8, 32), jnp.float32)
    ltab = ltab.at[:B * NPAD, :C].set(
        jnp.pad(class_logits, ((0, 0), (0, NPAD - N), (0, 0))).reshape(B * NPAD, C))

    orow_o, gath_o = _build_sc_nms()(d_t, p_t, l_t, ltab)
    orow = orow_o.reshape(B, OUTP, L)
    boxes_out = orow[:, :MAX_OUT, 0:5]
    scores_out = orow[:, :MAX_OUT, 5:7]
    ids_out = orow[:, :MAX_OUT, 7:9].astype(jnp.int32)
    logits_out = jnp.concatenate(
        [gath_o[:, :MAX_OUT, :C], orow[:, :MAX_OUT, 4:5]], axis=-1)
    return boxes_out, scores_out, ids_out, logits_out
